# baseline (speedup 1.0000x reference)
import functools

import jax
import jax.numpy as jnp
from jax import lax
from jax.experimental import pallas as pl
from jax.experimental.pallas import tpu as pltpu
from jax.experimental.pallas import tpu_sc as plsc

_NC = 2
_NS = 16
_NW = _NC * _NS
_SUB = 8
_CHUNK = 256


def _extract(vec, i):
    return jnp.squeeze(lax.slice(vec, (i,), (i + 1,)))


@functools.partial(jax.jit, static_argnums=(4, 5, 6))
def _gather_sc(uidx, iidx, ut3, it3, V, B, D):
    b_per_w = B // _NW
    n_chunks = b_per_w // _CHUNK
    c_slabs = _CHUNK // _SUB
    mesh = plsc.VectorSubcoreMesh(core_axis_name="c", subcore_axis_name="s")

    @functools.partial(
        pl.kernel,
        mesh=mesh,
        compiler_params=pltpu.CompilerParams(use_tc_tiling_on_sc=True),
        out_type=[
            jax.ShapeDtypeStruct((B // _SUB, _SUB, D), jnp.float32),
            jax.ShapeDtypeStruct((B // _SUB, _SUB, D), jnp.float32),
        ],
        scratch_types=[
            pltpu.VMEM((b_per_w,), jnp.int32),
            pltpu.VMEM((b_per_w,), jnp.int32),
            pltpu.VMEM((c_slabs, _SUB, D), jnp.float32),
            pltpu.VMEM((c_slabs, _SUB, D), jnp.float32),
            pltpu.SemaphoreType.DMA,
            pltpu.SemaphoreType.DMA,
        ],
    )
    def k(uidx_hbm, iidx_hbm, ut_hbm, it_hbm, ue_hbm, ie_hbm,
          uix, iix, ubuf, ibuf, sem_u, sem_i):
        wid = lax.axis_index("s") * _NC + lax.axis_index("c")
        base = wid * b_per_w

        pltpu.sync_copy(uidx_hbm.at[pl.ds(base, b_per_w)], uix)
        pltpu.sync_copy(iidx_hbm.at[pl.ds(base, b_per_w)], iix)

        def issue_rows(tab_hbm, ixv, buf, sem, h):
            def body(g, carry):
                vec = ixv[pl.ds(h * _CHUNK + g * 16, 16)]
                for rr in range(16):
                    r = _extract(vec, rr)
                    q = lax.shift_right_logical(r, 3)
                    s = lax.bitwise_and(r, 7)
                    pltpu.async_copy(
                        tab_hbm.at[q, s],
                        buf.at[2 * g + rr // _SUB, rr % _SUB],
                        sem)
                return carry
            lax.fori_loop(0, _CHUNK // 16, body, 0)

        for h in range(n_chunks):
            issue_rows(ut_hbm, uix, ubuf, sem_u, h)
            issue_rows(it_hbm, iix, ibuf, sem_i, h)
            pltpu.make_async_copy(
                ut_hbm.at[pl.ds(0, c_slabs)], ubuf, sem_u).wait()
            pltpu.make_async_copy(
                it_hbm.at[pl.ds(0, c_slabs)], ibuf, sem_i).wait()
            out_off = wid * (b_per_w // _SUB) + h * c_slabs
            pltpu.sync_copy(ubuf, ue_hbm.at[pl.ds(out_off, c_slabs)])
            pltpu.sync_copy(ibuf, ie_hbm.at[pl.ds(out_off, c_slabs)])

    return k(uidx, iidx, ut3, it3)


def _dot_body(u_ref, i_ref, b_ref, o_ref):
    o_ref[...] = (
        jnp.sum(u_ref[...] * i_ref[...], axis=1, keepdims=True) + b_ref[0]
    )


@functools.partial(jax.jit, static_argnums=(3, 4, 5))
def _dot_tc(ue, ie, bias, B, D, blk):
    return pl.pallas_call(
        _dot_body,
        grid=(B // blk,),
        in_specs=[
            pl.BlockSpec((blk, D), lambda i: (i, 0)),
            pl.BlockSpec((blk, D), lambda i: (i, 0)),
            pl.BlockSpec(memory_space=pltpu.SMEM),
        ],
        out_specs=pl.BlockSpec((blk, 1), lambda i: (i, 0)),
        out_shape=jax.ShapeDtypeStruct((B, 1), jnp.float32),
    )(ue, ie, bias)


def kernel(x, user_table, item_table, bias):
    B = x.shape[0]
    V, D = user_table.shape
    uidx = x[:, 0]
    iidx = x[:, 1]
    ue3, ie3 = _gather_sc(
        uidx, iidx,
        user_table.reshape(V // _SUB, _SUB, D),
        item_table.reshape(V // _SUB, _SUB, D),
        V, B, D)
    ue = ue3.reshape(B, D)
    ie = ie3.reshape(B, D)
    out = _dot_tc(ue, ie, bias, B, D, 2048)
    return out, ue, ie
